# baseline (speedup 1.0000x reference)
import jax
import jax.numpy as jnp
from jax import lax
from jax.experimental import pallas as pl
from jax.experimental.pallas import tpu as pltpu
from jax.experimental.pallas import tpu_sc as plsc

N_EDGES = 320000
D_FEAT = 128
N_SEG = 10000

NC = 2
NS = 16
NW = NC * NS
ROWS_PER_W = N_EDGES // NW
CHUNK = 80
NCHUNK = ROWS_PER_W // CHUNK
SUB = 80
NSUB = CHUNK // SUB
WCHUNK = 80
NWCHUNK = N_SEG // WCHUNK
WPT = (NWCHUNK + NS - 1) // NS


def _sc_partial_kernel(x_hbm, ids_hbm, part_hbm, acc, xbuf0, xbuf1, xbuf2,
                       idxbuf, sem0, sem1, sem2,
                       isem, ssem0, ssem1, ssem2):
    c = lax.axis_index("c")
    s = lax.axis_index("s")
    wid = c * NS + s
    base = wid * ROWS_PER_W

    xbufs = (xbuf0, xbuf1, xbuf2)
    xsems = (sem0, sem1, sem2)
    ssems = (ssem0, ssem1, ssem2)
    NSLOT = 3

    def x_desc(g):
        slot = (g + 1) % NSLOT
        r = base + g * CHUNK
        return pltpu.make_async_copy(x_hbm.at[pl.ds(r, CHUNK)],
                                     xbufs[slot], xsems[slot])

    def ids_desc(g):
        r = base + g * CHUNK
        return pltpu.make_async_copy(ids_hbm.at[pl.ds(r, CHUNK)],
                                     idxbuf.at[g], isem)

    def scat_desc(g):
        slot = (g + 1) % NSLOT
        return pltpu.make_async_copy(xbufs[slot],
                                     acc.at[idxbuf.at[g]], ssems[slot])

    for g in range(NCHUNK):
        ids_desc(g).start()

    for g in range(min(NSLOT - 1, NCHUNK)):
        x_desc(g).start()

    zero = jnp.zeros((16,), jnp.float32)

    def zero_row(i, carry):
        for l in range(D_FEAT // 16):
            xbuf0[i, pl.ds(l * 16, 16)] = zero
        return carry

    lax.fori_loop(0, WCHUNK, zero_row, 0)
    for k in range(WPT):
        cid = k * NS + s

        @pl.when(cid < NWCHUNK)
        def _():
            r = pl.multiple_of(cid * WCHUNK, 8)
            pltpu.sync_copy(xbuf0.at[pl.ds(0, WCHUNK)],
                            acc.at[pl.ds(r, WCHUNK)])

    plsc.subcore_barrier()

    for g in range(NCHUNK):
        ids_desc(g).wait()

    for g in range(NCHUNK):
        nxt = g + NSLOT - 1
        if nxt < NCHUNK:
            if g >= 1:
                scat_desc(g - 1).wait()
            x_desc(nxt).start()
        x_desc(g).wait()
        scat_desc(g).start(add=True)
    for g in range(max(NCHUNK - NSLOT, 0), NCHUNK):
        scat_desc(g).wait()

    plsc.subcore_barrier()

    def rd_desc(k, cid, sl):
        r = pl.multiple_of(cid * WCHUNK, 8)
        return pltpu.make_async_copy(acc.at[pl.ds(r, WCHUNK)],
                                     xbufs[sl], xsems[sl])

    def wr_desc(k, cid, sl):
        r = pl.multiple_of(cid * WCHUNK, 8)
        return pltpu.make_async_copy(xbufs[sl], part_hbm.at[c, pl.ds(r, WCHUNK)],
                                     ssems[sl])

    for k in range(WPT):
        cid = k * NS + s
        sl = k % 2
        if k >= 2:
            pcid = (k - 2) * NS + s

            @pl.when(pcid < NWCHUNK)
            def _():
                wr_desc(k - 2, pcid, sl).wait()

        @pl.when(cid < NWCHUNK)
        def _():
            rd_desc(k, cid, sl).start()
            rd_desc(k, cid, sl).wait()
            wr_desc(k, cid, sl).start()

    for k in range(max(WPT - 2, 0), WPT):
        cid = k * NS + s
        sl = k % 2

        @pl.when(cid < NWCHUNK)
        def _():
            wr_desc(k, cid, sl).wait()


def _combine_body(a_ref, b_ref, o_ref):
    o_ref[...] = a_ref[0] + b_ref[0]


@jax.jit
def _run(x, ids):
    mesh = plsc.VectorSubcoreMesh(core_axis_name="c", subcore_axis_name="s")
    part = pl.kernel(
        _sc_partial_kernel,
        mesh=mesh,
        out_type=jax.ShapeDtypeStruct((NC, N_SEG, D_FEAT), jnp.float32),
        scratch_types=[
            pltpu.VMEM_SHARED((N_SEG, D_FEAT), jnp.float32),
            pltpu.VMEM((CHUNK, D_FEAT), jnp.float32),
            pltpu.VMEM((CHUNK, D_FEAT), jnp.float32),
            pltpu.VMEM((CHUNK, D_FEAT), jnp.float32),
            pltpu.VMEM((NCHUNK, SUB), jnp.int32),
        ] + [pltpu.SemaphoreType.DMA] * 7,
    )(x, ids)

    blk = 2000
    out = pl.pallas_call(
        _combine_body,
        grid=(N_SEG // blk,),
        in_specs=[
            pl.BlockSpec((1, blk, D_FEAT), lambda i: (0, i, 0)),
            pl.BlockSpec((1, blk, D_FEAT), lambda i: (1, i, 0)),
        ],
        out_specs=pl.BlockSpec((blk, D_FEAT), lambda i: (i, 0)),
        out_shape=jax.ShapeDtypeStruct((N_SEG, D_FEAT), jnp.float32),
    )(part, part)
    return out


def kernel(x, segment_ids, num_segments):
    ids = segment_ids.astype(jnp.int32)
    return _run(x, ids)


# speedup vs baseline: 1.0151x; 1.0151x over previous
import jax
import jax.numpy as jnp
from jax import lax
from jax.experimental import pallas as pl
from jax.experimental.pallas import tpu as pltpu
from jax.experimental.pallas import tpu_sc as plsc

N_EDGES = 320000
D_FEAT = 128
N_SEG = 10000

NC = 2
NS = 16
NW = NC * NS
ROWS_PER_W = N_EDGES // NW
CHUNK = 80
NCHUNK = ROWS_PER_W // CHUNK
SUB = 80
NSUB = CHUNK // SUB
WCHUNK = 80
NWCHUNK = N_SEG // WCHUNK
WPT = (NWCHUNK + NS - 1) // NS


def _sc_partial_kernel(x_hbm, ids_hbm, part_hbm, acc, xbuf0, xbuf1, xbuf2,
                       xbuf3, idx0, idx1, idx2, idx3, sem0, sem1, sem2, sem3,
                       isem0, isem1, isem2, isem3, ssem0, ssem1, ssem2, ssem3):
    c = lax.axis_index("c")
    s = lax.axis_index("s")
    wid = c * NS + s
    base = wid * ROWS_PER_W

    xbufs = (xbuf0, xbuf1, xbuf2, xbuf3)
    idxs = (idx0, idx1, idx2, idx3)
    xsems = (sem0, sem1, sem2, sem3)
    isems = (isem0, isem1, isem2, isem3)
    ssems = (ssem0, ssem1, ssem2, ssem3)
    NSLOT = 4

    def x_desc(g):
        slot = (g + 1) % NSLOT
        r = base + g * CHUNK
        return pltpu.make_async_copy(x_hbm.at[pl.ds(r, CHUNK)],
                                     xbufs[slot], xsems[slot])

    def ids_desc(g):
        slot = (g + 1) % NSLOT
        r = base + g * CHUNK
        return pltpu.make_async_copy(ids_hbm.at[pl.ds(r, CHUNK)],
                                     idxs[slot].at[0], isems[slot])

    def scat_desc(g):
        slot = (g + 1) % NSLOT
        return pltpu.make_async_copy(xbufs[slot],
                                     acc.at[idxs[slot].at[0]], ssems[slot])

    for g in range(min(NSLOT - 2, NCHUNK)):
        x_desc(g).start()
        ids_desc(g).start()

    zero = jnp.zeros((16,), jnp.float32)

    def zero_row(i, carry):
        for l in range(D_FEAT // 16):
            xbuf0[i, pl.ds(l * 16, 16)] = zero
        return carry

    lax.fori_loop(0, WCHUNK, zero_row, 0)
    for k in range(WPT):
        cid = k * NS + s

        @pl.when(cid < NWCHUNK)
        def _():
            r = pl.multiple_of(cid * WCHUNK, 8)
            pltpu.sync_copy(xbuf0.at[pl.ds(0, WCHUNK)],
                            acc.at[pl.ds(r, WCHUNK)])

    plsc.subcore_barrier()

    for g in range(NCHUNK):
        nxt = g + NSLOT - 2
        if nxt < NCHUNK:
            if g >= 2:
                scat_desc(g - 2).wait()
            x_desc(nxt).start()
            ids_desc(nxt).start()
        x_desc(g).wait()
        ids_desc(g).wait()
        scat_desc(g).start(add=True)
    for g in range(max(NCHUNK - NSLOT, 0), NCHUNK):
        scat_desc(g).wait()

    plsc.subcore_barrier()

    def rd_desc(k, cid, sl):
        r = pl.multiple_of(cid * WCHUNK, 8)
        return pltpu.make_async_copy(acc.at[pl.ds(r, WCHUNK)],
                                     xbufs[sl], xsems[sl])

    def wr_desc(k, cid, sl):
        r = pl.multiple_of(cid * WCHUNK, 8)
        return pltpu.make_async_copy(xbufs[sl], part_hbm.at[c, pl.ds(r, WCHUNK)],
                                     ssems[sl])

    for k in range(WPT):
        cid = k * NS + s
        sl = k % 2
        if k >= 2:
            pcid = (k - 2) * NS + s

            @pl.when(pcid < NWCHUNK)
            def _():
                wr_desc(k - 2, pcid, sl).wait()

        @pl.when(cid < NWCHUNK)
        def _():
            rd_desc(k, cid, sl).start()
            rd_desc(k, cid, sl).wait()
            wr_desc(k, cid, sl).start()

    for k in range(max(WPT - 2, 0), WPT):
        cid = k * NS + s
        sl = k % 2

        @pl.when(cid < NWCHUNK)
        def _():
            wr_desc(k, cid, sl).wait()


def _combine_body(a_ref, b_ref, o_ref):
    o_ref[...] = a_ref[0] + b_ref[0]


@jax.jit
def _run(x, ids):
    mesh = plsc.VectorSubcoreMesh(core_axis_name="c", subcore_axis_name="s")
    part = pl.kernel(
        _sc_partial_kernel,
        mesh=mesh,
        out_type=jax.ShapeDtypeStruct((NC, N_SEG, D_FEAT), jnp.float32),
        scratch_types=[
            pltpu.VMEM_SHARED((N_SEG, D_FEAT), jnp.float32),
            pltpu.VMEM((CHUNK, D_FEAT), jnp.float32),
            pltpu.VMEM((CHUNK, D_FEAT), jnp.float32),
            pltpu.VMEM((CHUNK, D_FEAT), jnp.float32),
            pltpu.VMEM((CHUNK, D_FEAT), jnp.float32),
            pltpu.VMEM((NSUB, SUB), jnp.int32),
            pltpu.VMEM((NSUB, SUB), jnp.int32),
            pltpu.VMEM((NSUB, SUB), jnp.int32),
            pltpu.VMEM((NSUB, SUB), jnp.int32),
        ] + [pltpu.SemaphoreType.DMA] * 12,
    )(x, ids)

    blk = 2000
    out = pl.pallas_call(
        _combine_body,
        grid=(N_SEG // blk,),
        in_specs=[
            pl.BlockSpec((1, blk, D_FEAT), lambda i: (0, i, 0)),
            pl.BlockSpec((1, blk, D_FEAT), lambda i: (1, i, 0)),
        ],
        out_specs=pl.BlockSpec((blk, D_FEAT), lambda i: (i, 0)),
        out_shape=jax.ShapeDtypeStruct((N_SEG, D_FEAT), jnp.float32),
    )(part, part)
    return out


def kernel(x, segment_ids, num_segments):
    ids = segment_ids.astype(jnp.int32)
    return _run(x, ids)
